# Initial kernel scaffold; baseline (speedup 1.0000x reference)
import jax
import jax.numpy as jnp
from jax.experimental import pallas as pl


def kernel(x, table, W, b, bn_gamma, bn_beta, bn_mean, bn_var, ln_gamma, ln_beta):
    raise NotImplementedError("write your pallas kernel here")


# speedup vs baseline: 2.3963x; 2.3963x over previous
import functools

import jax
import jax.numpy as jnp
from jax import lax
from jax.experimental import pallas as pl
from jax.experimental.pallas import tpu as pltpu
from jax.experimental.pallas import tpu_sc as plsc

NC = 2
NS = 16
NW = NC * NS

CB = 16
IDX_MINOR = 100


def _sc_gather_sum(x2, table, B, L, D):
    b_per_w = B // NW
    n_chunks = b_per_w // CB
    rows_per_chunk = CB * L
    n_sub = rows_per_chunk // IDX_MINOR
    idx_rows_per_chunk = n_sub

    mesh = plsc.VectorSubcoreMesh(core_axis_name="c", subcore_axis_name="s")

    @functools.partial(
        pl.kernel,
        mesh=mesh,
        compiler_params=pltpu.CompilerParams(use_tc_tiling_on_sc=False),
        out_type=jax.ShapeDtypeStruct((B, D), jnp.float32),
        scratch_types=[
            pltpu.VMEM((n_sub, IDX_MINOR), jnp.int32),
            pltpu.VMEM((rows_per_chunk, D), jnp.float32),
            pltpu.VMEM((CB, D), jnp.float32),
            pltpu.SemaphoreType.DMA,
        ],
    )
    def k(x2_hbm, table_hbm, out_hbm, idx_v, rows_v, acc_v, sem):
        wid = lax.axis_index("s") * NC + lax.axis_index("c")
        b_base = wid * b_per_w
        ir_base = b_base * L // IDX_MINOR

        def chunk(ci, _):
            b0 = pl.multiple_of(b_base + ci * CB, CB)
            ir0 = pl.multiple_of(ir_base + ci * idx_rows_per_chunk,
                                 idx_rows_per_chunk)
            pltpu.sync_copy(x2_hbm.at[pl.ds(ir0, idx_rows_per_chunk)], idx_v)
            cps = [
                pltpu.async_copy(
                    table_hbm.at[idx_v.at[j]],
                    rows_v.at[pl.ds(j * IDX_MINOR, IDX_MINOR)],
                    sem,
                )
                for j in range(n_sub)
            ]
            for cp in cps:
                cp.wait()
            for i in range(CB):
                r0 = i * L

                def lbody(l, carry):
                    row = r0 + l
                    return tuple(
                        carry[t] + rows_v[row, pl.ds(t * 16, 16)]
                        for t in range(D // 16)
                    )

                init = tuple(
                    rows_v[r0, pl.ds(t * 16, 16)] for t in range(D // 16)
                )
                ss = lax.fori_loop(1, L, lbody, init)
                for t in range(D // 16):
                    acc_v[i, pl.ds(t * 16, 16)] = ss[t]
            pltpu.sync_copy(acc_v, out_hbm.at[pl.ds(b0, CB)])
            return 0

        lax.fori_loop(0, n_chunks, chunk, 0)

    return k(x2, table)


def _tc_head(h, W2, b2, ln_gamma, ln_beta):
    B, D = h.shape
    bm = 1024

    def body(h_ref, w_ref, b_ref, g_ref, be_ref, o_ref):
        y = jnp.dot(h_ref[...], w_ref[...],
                    preferred_element_type=jnp.float32) + b_ref[...]
        mu = jnp.mean(y, axis=-1, keepdims=True)
        var = jnp.mean(jnp.square(y - mu), axis=-1, keepdims=True)
        o_ref[...] = (y - mu) * lax.rsqrt(var + 1e-3) * g_ref[...] + be_ref[...]

    return pl.pallas_call(
        body,
        grid=(B // bm,),
        in_specs=[
            pl.BlockSpec((bm, D), lambda i: (i, 0)),
            pl.BlockSpec((D, D), lambda i: (0, 0)),
            pl.BlockSpec((1, D), lambda i: (0, 0)),
            pl.BlockSpec((1, D), lambda i: (0, 0)),
            pl.BlockSpec((1, D), lambda i: (0, 0)),
        ],
        out_specs=pl.BlockSpec((bm, D), lambda i: (i, 0)),
        out_shape=jax.ShapeDtypeStruct((B, D), jnp.float32),
    )(h, W2, b2, ln_gamma, ln_beta)


def kernel(x, table, W, b, bn_gamma, bn_beta, bn_mean, bn_var, ln_gamma,
           ln_beta):
    B, L = x.shape
    V, D = table.shape
    s = bn_gamma * lax.rsqrt(bn_var + 1e-3)
    W2 = W * s[None, :] * (1.0 / L)
    b2 = (b - bn_mean) * s + bn_beta
    x2 = x.astype(jnp.int32).reshape(B * L // IDX_MINOR, IDX_MINOR)
    h = _sc_gather_sum(x2, table, B, L, D)
    return _tc_head(h, W2, b2.reshape(1, D), ln_gamma.reshape(1, D),
                    ln_beta.reshape(1, D))


# speedup vs baseline: 2.3980x; 1.0007x over previous
import functools

import jax
import jax.numpy as jnp
from jax import lax
from jax.experimental import pallas as pl
from jax.experimental.pallas import tpu as pltpu
from jax.experimental.pallas import tpu_sc as plsc

NC = 2
NS = 16
NW = NC * NS

CB = 16


def _sc_gather_sum(x, table, B, L, D):
    b_per_w = B // NW
    n_chunks = b_per_w // CB
    rows_per_chunk = CB * L

    mesh = plsc.VectorSubcoreMesh(core_axis_name="c", subcore_axis_name="s")

    @functools.partial(
        pl.kernel,
        mesh=mesh,
        compiler_params=pltpu.CompilerParams(use_tc_tiling_on_sc=False),
        out_type=jax.ShapeDtypeStruct((B, D), jnp.float32),
        scratch_types=[
            pltpu.VMEM((CB, L), jnp.int32),
            pltpu.VMEM((rows_per_chunk, D), jnp.float32),
            pltpu.VMEM((CB, D), jnp.float32),
            pltpu.SemaphoreType.DMA,
        ],
    )
    def k(x_hbm, table_hbm, out_hbm, idx_v, rows_v, acc_v, sem):
        wid = lax.axis_index("s") * NC + lax.axis_index("c")
        b_base = wid * b_per_w

        def chunk(ci, _):
            b0 = pl.multiple_of(b_base + ci * CB, CB)
            pltpu.sync_copy(x_hbm.at[pl.ds(b0, CB)], idx_v)
            cps = [
                pltpu.async_copy(
                    table_hbm.at[idx_v.at[j]],
                    rows_v.at[pl.ds(j * L, L)],
                    sem,
                )
                for j in range(CB)
            ]
            for cp in cps:
                cp.wait()
            for i in range(CB):
                r0 = i * L

                def lbody(l, carry):
                    row = r0 + l
                    return tuple(
                        carry[t] + rows_v[row, pl.ds(t * 16, 16)]
                        for t in range(D // 16)
                    )

                init = tuple(
                    rows_v[r0, pl.ds(t * 16, 16)] for t in range(D // 16)
                )
                ss = lax.fori_loop(1, L, lbody, init)
                for t in range(D // 16):
                    acc_v[i, pl.ds(t * 16, 16)] = ss[t]
            pltpu.sync_copy(acc_v, out_hbm.at[pl.ds(b0, CB)])
            return 0

        lax.fori_loop(0, n_chunks, chunk, 0)

    return k(x, table)


def _tc_head(h, W2, b2, ln_gamma, ln_beta):
    B, D = h.shape
    bm = 1024

    def body(h_ref, w_ref, b_ref, g_ref, be_ref, o_ref):
        y = jnp.dot(h_ref[...], w_ref[...],
                    preferred_element_type=jnp.float32) + b_ref[...]
        mu = jnp.mean(y, axis=-1, keepdims=True)
        var = jnp.mean(jnp.square(y - mu), axis=-1, keepdims=True)
        o_ref[...] = (y - mu) * lax.rsqrt(var + 1e-3) * g_ref[...] + be_ref[...]

    return pl.pallas_call(
        body,
        grid=(B // bm,),
        in_specs=[
            pl.BlockSpec((bm, D), lambda i: (i, 0)),
            pl.BlockSpec((D, D), lambda i: (0, 0)),
            pl.BlockSpec((1, D), lambda i: (0, 0)),
            pl.BlockSpec((1, D), lambda i: (0, 0)),
            pl.BlockSpec((1, D), lambda i: (0, 0)),
        ],
        out_specs=pl.BlockSpec((bm, D), lambda i: (i, 0)),
        out_shape=jax.ShapeDtypeStruct((B, D), jnp.float32),
    )(h, W2, b2, ln_gamma, ln_beta)


def kernel(x, table, W, b, bn_gamma, bn_beta, bn_mean, bn_var, ln_gamma,
           ln_beta):
    B, L = x.shape
    V, D = table.shape
    s = bn_gamma * lax.rsqrt(bn_var + 1e-3)
    W2 = W * s[None, :] * (1.0 / L)
    b2 = (b - bn_mean) * s + bn_beta
    h = _sc_gather_sum(x.astype(jnp.int32), table, B, L, D)
    return _tc_head(h, W2, b2.reshape(1, D), ln_gamma.reshape(1, D),
                    ln_beta.reshape(1, D))


# speedup vs baseline: 4.1117x; 1.7146x over previous
import functools

import numpy as np
import jax
import jax.numpy as jnp
from jax import lax
from jax.experimental import pallas as pl
from jax.experimental.pallas import tpu as pltpu
from jax.experimental.pallas import tpu_sc as plsc

NC = 2
NS = 16
NW = NC * NS

CB = 16
TBN = 4096


def _bf16_bits(x):
    xb = lax.bitcast_convert_type(x, jnp.uint32)
    return (xb + jnp.uint32(0x7FFF) + ((xb >> 16) & jnp.uint32(1))) >> 16


def _tc_pack_table(tableT):
    Dd, V = tableT.shape
    nblk = (V + 4 * TBN - 1) // (4 * TBN)
    H = Dd // 2
    last_blk = (V + TBN - 1) // TBN - 1

    def body(a_ref, b_ref, c_ref, d_ref, o_ref):
        for q, ref in enumerate((a_ref, b_ref, c_ref, d_ref)):
            tr = jnp.swapaxes(ref[...], 0, 1)
            lo = _bf16_bits(tr[:, 0:H])
            hi = _bf16_bits(tr[:, H:2 * H])
            packed = lax.bitcast_convert_type(lo | (hi << 16), jnp.int32)
            o_ref[:, q * H:(q + 1) * H] = packed

    out = pl.pallas_call(
        body,
        grid=(nblk,),
        in_specs=[
            pl.BlockSpec((Dd, TBN),
                         lambda i, q=q: (0, jnp.minimum(4 * i + q, last_blk)))
            for q in range(4)
        ],
        out_specs=pl.BlockSpec((TBN, 4 * H), lambda i: (i, 0)),
        out_shape=jax.ShapeDtypeStruct((nblk * TBN, 4 * H), jnp.int32),
    )(tableT, tableT, tableT, tableT)
    return out.reshape(4 * nblk * TBN, H)


def _remap_idx(x):
    s = TBN.bit_length() - 1
    return (x & ~(4 * TBN - 1)) | ((x & (TBN - 1)) << 2) | ((x >> s) & 3)


def _sc_gather_sum(x, table, B, L, D):
    b_per_w = B // NW
    n_chunks = b_per_w // CB
    rows_per_chunk = CB * L
    npair = D // 32
    HW = D // 2

    mesh = plsc.VectorSubcoreMesh(core_axis_name="c", subcore_axis_name="s")

    @functools.partial(
        pl.kernel,
        mesh=mesh,
        compiler_params=pltpu.CompilerParams(
            use_tc_tiling_on_sc=False, needs_layout_passes=False),
        out_type=jax.ShapeDtypeStruct((B, D), jnp.float32),
        scratch_types=[
            pltpu.VMEM((CB, L), jnp.int32),
            pltpu.VMEM((CB, L), jnp.int32),
            pltpu.VMEM((rows_per_chunk, HW), jnp.int32),
            pltpu.VMEM((rows_per_chunk, HW), jnp.int32),
            pltpu.VMEM((CB, D), jnp.float32),
            pltpu.SemaphoreType.DMA,
            pltpu.SemaphoreType.DMA,
        ],
    )
    def k(x_hbm, table_hbm, out_hbm, idxA, idxB, rowsA, rowsB, acc_v,
          semA, semB):
        wid = lax.axis_index("s") * NC + lax.axis_index("c")
        b_base = wid * b_per_w

        def issue(ci, idxr, rowsr, sem):
            b0 = pl.multiple_of(b_base + ci * CB, CB)
            pltpu.sync_copy(x_hbm.at[pl.ds(b0, CB)], idxr)
            return [
                pltpu.async_copy(
                    table_hbm.at[idxr.at[j]],
                    rowsr.at[pl.ds(j * L, L)],
                    sem,
                )
                for j in range(CB)
            ]

        def reduce_out(ci, rows):
            mask_hi = jnp.full((16,), -65536, jnp.int32)

            def halves(row, p):
                v = rows[row, pl.ds(p * 16, 16)]
                lo = lax.bitcast_convert_type(lax.shift_left(v, 16),
                                              jnp.float32)
                hi = lax.bitcast_convert_type(v & mask_hi, jnp.float32)
                return lo, hi

            for i in range(CB):
                r0 = i * L
                init = []
                for p in range(npair):
                    lo, hi = halves(r0, p)
                    init += [lo, hi]

                def lbody(u, carry):
                    base = r0 + 1 + u * 7
                    for q in range(7):
                        new = []
                        for p in range(npair):
                            lo, hi = halves(base + q, p)
                            new += [carry[2 * p] + lo, carry[2 * p + 1] + hi]
                        carry = tuple(new)
                    return carry

                ss = lax.fori_loop(0, (L - 1) // 7, lbody, tuple(init))
                for t in range(2 * npair):
                    acc_v[i, pl.ds(t * 16, 16)] = ss[t]
            b0 = pl.multiple_of(b_base + ci * CB, CB)
            pltpu.sync_copy(acc_v, out_hbm.at[pl.ds(b0, CB)])

        def body(g, _):
            ci = g * 2
            cpsA = issue(ci, idxA, rowsA, semA)
            for cp in cpsA:
                cp.wait()
            cpsB = issue(ci + 1, idxB, rowsB, semB)
            reduce_out(ci, rowsA)
            for cp in cpsB:
                cp.wait()
            reduce_out(ci + 1, rowsB)
            return 0

        lax.fori_loop(0, n_chunks // 2, body, 0)

    return k(x, table)


def _tc_head(h, W2, b2, ln_gamma, ln_beta):
    B, D = h.shape
    bm = 1024

    def body(h_ref, w_ref, b_ref, g_ref, be_ref, o_ref):
        y = jnp.dot(h_ref[...], w_ref[...],
                    preferred_element_type=jnp.float32) + b_ref[...]
        mu = jnp.mean(y, axis=-1, keepdims=True)
        var = jnp.mean(jnp.square(y - mu), axis=-1, keepdims=True)
        o_ref[...] = (y - mu) * lax.rsqrt(var + 1e-3) * g_ref[...] + be_ref[...]

    return pl.pallas_call(
        body,
        grid=(B // bm,),
        in_specs=[
            pl.BlockSpec((bm, D), lambda i: (i, 0)),
            pl.BlockSpec((D, D), lambda i: (0, 0)),
            pl.BlockSpec((1, D), lambda i: (0, 0)),
            pl.BlockSpec((1, D), lambda i: (0, 0)),
            pl.BlockSpec((1, D), lambda i: (0, 0)),
        ],
        out_specs=pl.BlockSpec((bm, D), lambda i: (i, 0)),
        out_shape=jax.ShapeDtypeStruct((B, D), jnp.float32),
    )(h, W2, b2, ln_gamma, ln_beta)


def _pair_perm(D):
    perm = []
    for p in range(D // 32):
        perm += list(range(p * 16, p * 16 + 16))
        perm += list(range(p * 16 + 32, p * 16 + 48))
    return np.array(perm)


def kernel(x, table, W, b, bn_gamma, bn_beta, bn_mean, bn_var, ln_gamma,
           ln_beta):
    B, L = x.shape
    V, D = table.shape
    s = bn_gamma * lax.rsqrt(bn_var + 1e-3)
    W2 = W * s[None, :] * (1.0 / L)
    b2 = (b - bn_mean) * s + bn_beta
    W2p = W2[_pair_perm(D), :]
    table_pk = _tc_pack_table(jnp.swapaxes(table, 0, 1))
    x2 = _remap_idx(x.astype(jnp.int32))
    h = _sc_gather_sum(x2, table_pk, B, L, D)
    return _tc_head(h, W2p, b2.reshape(1, D), ln_gamma.reshape(1, D),
                    ln_beta.reshape(1, D))


# speedup vs baseline: 4.1530x; 1.0100x over previous
import functools

import numpy as np
import jax
import jax.numpy as jnp
from jax import lax
from jax.experimental import pallas as pl
from jax.experimental.pallas import tpu as pltpu
from jax.experimental.pallas import tpu_sc as plsc

NC = 2
NS = 16
NW = NC * NS

CB = 16
TBN = 4096


def _bf16_bits(x):
    h = lax.bitcast_convert_type(x.astype(jnp.bfloat16), jnp.uint16)
    return h.astype(jnp.uint32)


def _tc_pack_table(tableT):
    Dd, V = tableT.shape
    nblk = (V + 4 * TBN - 1) // (4 * TBN)
    H = Dd // 2
    last_blk = (V + TBN - 1) // TBN - 1

    def body(a_ref, b_ref, c_ref, d_ref, o_ref):
        rows = lax.broadcasted_iota(jnp.int32, (Dd, Dd), 0)
        cols = lax.broadcasted_iota(jnp.int32, (Dd, Dd), 1)
        eye = (rows == cols).astype(jnp.float32)
        for q, ref in enumerate((a_ref, b_ref, c_ref, d_ref)):
            tr = lax.dot_general(ref[...], eye, (((0,), (0,)), ((), ())),
                                 preferred_element_type=jnp.float32)
            lo = _bf16_bits(tr[:, 0:H])
            hi = _bf16_bits(tr[:, H:2 * H])
            packed = lax.bitcast_convert_type(lo | (hi << 16), jnp.int32)
            o_ref[:, q * H:(q + 1) * H] = packed

    out = pl.pallas_call(
        body,
        grid=(nblk,),
        in_specs=[
            pl.BlockSpec((Dd, TBN),
                         lambda i, q=q: (0, jnp.minimum(4 * i + q, last_blk)))
            for q in range(4)
        ],
        out_specs=pl.BlockSpec((TBN, 4 * H), lambda i: (i, 0)),
        out_shape=jax.ShapeDtypeStruct((nblk * TBN, 4 * H), jnp.int32),
    )(tableT, tableT, tableT, tableT)
    return out.reshape(4 * nblk * TBN, H)


def _remap_idx(x):
    s = TBN.bit_length() - 1
    return (x & ~(4 * TBN - 1)) | ((x & (TBN - 1)) << 2) | ((x >> s) & 3)


def _sc_gather_sum(x, table, B, L, D):
    b_per_w = B // NW
    n_chunks = b_per_w // CB
    rows_per_chunk = CB * L
    npair = D // 32
    HW = D // 2

    mesh = plsc.VectorSubcoreMesh(core_axis_name="c", subcore_axis_name="s")

    @functools.partial(
        pl.kernel,
        mesh=mesh,
        compiler_params=pltpu.CompilerParams(
            use_tc_tiling_on_sc=False, needs_layout_passes=False),
        out_type=jax.ShapeDtypeStruct((B, D), jnp.float32),
        scratch_types=[
            pltpu.VMEM((CB, L), jnp.int32),
            pltpu.VMEM((CB, L), jnp.int32),
            pltpu.VMEM((rows_per_chunk, HW), jnp.int32),
            pltpu.VMEM((rows_per_chunk, HW), jnp.int32),
            pltpu.VMEM((CB, D), jnp.float32),
            pltpu.SemaphoreType.DMA,
            pltpu.SemaphoreType.DMA,
        ],
    )
    def k(x_hbm, table_hbm, out_hbm, idxA, idxB, rowsA, rowsB, acc_v,
          semA, semB):
        wid = lax.axis_index("s") * NC + lax.axis_index("c")
        b_base = wid * b_per_w

        def issue(ci, idxr, rowsr, sem):
            b0 = pl.multiple_of(b_base + ci * CB, CB)
            pltpu.sync_copy(x_hbm.at[pl.ds(b0, CB)], idxr)
            return [
                pltpu.async_copy(
                    table_hbm.at[idxr.at[j]],
                    rowsr.at[pl.ds(j * L, L)],
                    sem,
                )
                for j in range(CB)
            ]

        def reduce_out(ci, rows):
            mask_hi = jnp.full((16,), -65536, jnp.int32)

            def halves(row, p):
                v = rows[row, pl.ds(p * 16, 16)]
                lo = lax.bitcast_convert_type(lax.shift_left(v, 16),
                                              jnp.float32)
                hi = lax.bitcast_convert_type(v & mask_hi, jnp.float32)
                return lo, hi

            for i in range(CB):
                r0 = i * L
                init = []
                for p in range(npair):
                    lo, hi = halves(r0, p)
                    init += [lo, hi]

                def lbody(u, carry):
                    base = r0 + 1 + u * 7
                    for q in range(7):
                        new = []
                        for p in range(npair):
                            lo, hi = halves(base + q, p)
                            new += [carry[2 * p] + lo, carry[2 * p + 1] + hi]
                        carry = tuple(new)
                    return carry

                ss = lax.fori_loop(0, (L - 1) // 7, lbody, tuple(init))
                for t in range(2 * npair):
                    acc_v[i, pl.ds(t * 16, 16)] = ss[t]
            b0 = pl.multiple_of(b_base + ci * CB, CB)
            pltpu.sync_copy(acc_v, out_hbm.at[pl.ds(b0, CB)])

        def body(g, _):
            ci = g * 2
            cpsA = issue(ci, idxA, rowsA, semA)
            for cp in cpsA:
                cp.wait()
            cpsB = issue(ci + 1, idxB, rowsB, semB)
            reduce_out(ci, rowsA)
            for cp in cpsB:
                cp.wait()
            reduce_out(ci + 1, rowsB)
            return 0

        lax.fori_loop(0, n_chunks // 2, body, 0)

    return k(x, table)


def _tc_head(h, W2, b2, ln_gamma, ln_beta):
    B, D = h.shape
    bm = 1024

    def body(h_ref, w_ref, b_ref, g_ref, be_ref, o_ref):
        y = jnp.dot(h_ref[...], w_ref[...],
                    preferred_element_type=jnp.float32) + b_ref[...]
        mu = jnp.mean(y, axis=-1, keepdims=True)
        var = jnp.mean(jnp.square(y - mu), axis=-1, keepdims=True)
        o_ref[...] = (y - mu) * lax.rsqrt(var + 1e-3) * g_ref[...] + be_ref[...]

    return pl.pallas_call(
        body,
        grid=(B // bm,),
        in_specs=[
            pl.BlockSpec((bm, D), lambda i: (i, 0)),
            pl.BlockSpec((D, D), lambda i: (0, 0)),
            pl.BlockSpec((1, D), lambda i: (0, 0)),
            pl.BlockSpec((1, D), lambda i: (0, 0)),
            pl.BlockSpec((1, D), lambda i: (0, 0)),
        ],
        out_specs=pl.BlockSpec((bm, D), lambda i: (i, 0)),
        out_shape=jax.ShapeDtypeStruct((B, D), jnp.float32),
    )(h, W2, b2, ln_gamma, ln_beta)


def _pair_perm(D):
    perm = []
    for p in range(D // 32):
        perm += list(range(p * 16, p * 16 + 16))
        perm += list(range(p * 16 + 32, p * 16 + 48))
    return np.array(perm)


def kernel(x, table, W, b, bn_gamma, bn_beta, bn_mean, bn_var, ln_gamma,
           ln_beta):
    B, L = x.shape
    V, D = table.shape
    s = bn_gamma * lax.rsqrt(bn_var + 1e-3)
    W2 = W * s[None, :] * (1.0 / L)
    b2 = (b - bn_mean) * s + bn_beta
    W2p = W2[_pair_perm(D), :]
    table_pk = _tc_pack_table(jnp.swapaxes(table, 0, 1))
    x2 = _remap_idx(x.astype(jnp.int32))
    h = _sc_gather_sum(x2, table_pk, B, L, D)
    return _tc_head(h, W2p, b2.reshape(1, D), ln_gamma.reshape(1, D),
                    ln_beta.reshape(1, D))


# speedup vs baseline: 4.1973x; 1.0107x over previous
import functools

import numpy as np
import jax
import jax.numpy as jnp
from jax import lax
from jax.experimental import pallas as pl
from jax.experimental.pallas import tpu as pltpu
from jax.experimental.pallas import tpu_sc as plsc

NC = 2
NS = 16
NW = NC * NS

CB = 16
TBN = 8192


def _bf16_bits(x):
    h = lax.bitcast_convert_type(x.astype(jnp.bfloat16), jnp.uint16)
    return h.astype(jnp.uint32)


def _tc_pack_table(tableT):
    Dd, V = tableT.shape
    nblk = (V + 4 * TBN - 1) // (4 * TBN)
    H = Dd // 2
    last_blk = (V + TBN - 1) // TBN - 1

    def body(a_ref, b_ref, c_ref, d_ref, o_ref):
        rows = lax.broadcasted_iota(jnp.int32, (Dd, Dd), 0)
        cols = lax.broadcasted_iota(jnp.int32, (Dd, Dd), 1)
        eye = (rows == cols).astype(jnp.float32)
        for q, ref in enumerate((a_ref, b_ref, c_ref, d_ref)):
            tr = lax.dot_general(ref[...], eye, (((0,), (0,)), ((), ())),
                                 preferred_element_type=jnp.float32)
            lo = _bf16_bits(tr[:, 0:H])
            hi = _bf16_bits(tr[:, H:2 * H])
            packed = lax.bitcast_convert_type(lo | (hi << 16), jnp.int32)
            o_ref[:, q * H:(q + 1) * H] = packed

    out = pl.pallas_call(
        body,
        grid=(nblk,),
        compiler_params=pltpu.CompilerParams(
            vmem_limit_bytes=100 * 1024 * 1024),
        in_specs=[
            pl.BlockSpec((Dd, TBN),
                         lambda i, q=q: (0, jnp.minimum(4 * i + q, last_blk)))
            for q in range(4)
        ],
        out_specs=pl.BlockSpec((TBN, 4 * H), lambda i: (i, 0)),
        out_shape=jax.ShapeDtypeStruct((nblk * TBN, 4 * H), jnp.int32),
    )(tableT, tableT, tableT, tableT)
    return out.reshape(4 * nblk * TBN, H)


def _remap_idx(x):
    s = TBN.bit_length() - 1
    return (x & ~(4 * TBN - 1)) | ((x & (TBN - 1)) << 2) | ((x >> s) & 3)


def _sc_gather_sum(x, table, B, L, D):
    b_per_w = B // NW
    n_chunks = b_per_w // CB
    rows_per_chunk = CB * L
    npair = D // 32
    HW = D // 2

    mesh = plsc.VectorSubcoreMesh(core_axis_name="c", subcore_axis_name="s")

    @functools.partial(
        pl.kernel,
        mesh=mesh,
        compiler_params=pltpu.CompilerParams(
            use_tc_tiling_on_sc=False, needs_layout_passes=False),
        out_type=jax.ShapeDtypeStruct((B, D), jnp.float32),
        scratch_types=[
            pltpu.VMEM((CB, L), jnp.int32),
            pltpu.VMEM((CB, L), jnp.int32),
            pltpu.VMEM((rows_per_chunk, HW), jnp.int32),
            pltpu.VMEM((rows_per_chunk, HW), jnp.int32),
            pltpu.VMEM((CB, D), jnp.float32),
            pltpu.SemaphoreType.DMA,
            pltpu.SemaphoreType.DMA,
        ],
    )
    def k(x_hbm, table_hbm, out_hbm, idxA, idxB, rowsA, rowsB, acc_v,
          semA, semB):
        wid = lax.axis_index("s") * NC + lax.axis_index("c")
        b_base = wid * b_per_w

        def issue(ci, idxr, rowsr, sem):
            b0 = pl.multiple_of(b_base + ci * CB, CB)
            pltpu.sync_copy(x_hbm.at[pl.ds(b0, CB)], idxr)
            return [
                pltpu.async_copy(
                    table_hbm.at[idxr.at[j]],
                    rowsr.at[pl.ds(j * L, L)],
                    sem,
                )
                for j in range(CB)
            ]

        def reduce_out(ci, rows):
            mask_hi = jnp.full((16,), -65536, jnp.int32)

            def halves(row, p):
                v = rows[row, pl.ds(p * 16, 16)]
                lo = lax.bitcast_convert_type(lax.shift_left(v, 16),
                                              jnp.float32)
                hi = lax.bitcast_convert_type(v & mask_hi, jnp.float32)
                return lo, hi

            for i in range(CB):
                r0 = i * L
                init = []
                for p in range(npair):
                    lo, hi = halves(r0, p)
                    init += [lo, hi]

                def lbody(u, carry):
                    base = r0 + 1 + u * 7
                    for q in range(7):
                        new = []
                        for p in range(npair):
                            lo, hi = halves(base + q, p)
                            new += [carry[2 * p] + lo, carry[2 * p + 1] + hi]
                        carry = tuple(new)
                    return carry

                ss = lax.fori_loop(0, (L - 1) // 7, lbody, tuple(init))
                for t in range(2 * npair):
                    acc_v[i, pl.ds(t * 16, 16)] = ss[t]
            b0 = pl.multiple_of(b_base + ci * CB, CB)
            pltpu.sync_copy(acc_v, out_hbm.at[pl.ds(b0, CB)])

        def body(g, _):
            ci = g * 2
            cpsA = issue(ci, idxA, rowsA, semA)
            for cp in cpsA:
                cp.wait()
            cpsB = issue(ci + 1, idxB, rowsB, semB)
            reduce_out(ci, rowsA)
            for cp in cpsB:
                cp.wait()
            reduce_out(ci + 1, rowsB)
            return 0

        lax.fori_loop(0, n_chunks // 2, body, 0)

    return k(x, table)


def _tc_head(h, W2, b2, ln_gamma, ln_beta):
    B, D = h.shape
    bm = 1024

    def body(h_ref, w_ref, b_ref, g_ref, be_ref, o_ref):
        y = jnp.dot(h_ref[...], w_ref[...],
                    preferred_element_type=jnp.float32) + b_ref[...]
        mu = jnp.mean(y, axis=-1, keepdims=True)
        var = jnp.mean(jnp.square(y - mu), axis=-1, keepdims=True)
        o_ref[...] = (y - mu) * lax.rsqrt(var + 1e-3) * g_ref[...] + be_ref[...]

    return pl.pallas_call(
        body,
        grid=(B // bm,),
        in_specs=[
            pl.BlockSpec((bm, D), lambda i: (i, 0)),
            pl.BlockSpec((D, D), lambda i: (0, 0)),
            pl.BlockSpec((1, D), lambda i: (0, 0)),
            pl.BlockSpec((1, D), lambda i: (0, 0)),
            pl.BlockSpec((1, D), lambda i: (0, 0)),
        ],
        out_specs=pl.BlockSpec((bm, D), lambda i: (i, 0)),
        out_shape=jax.ShapeDtypeStruct((B, D), jnp.float32),
    )(h, W2, b2, ln_gamma, ln_beta)


def _pair_perm(D):
    perm = []
    for p in range(D // 32):
        perm += list(range(p * 16, p * 16 + 16))
        perm += list(range(p * 16 + 32, p * 16 + 48))
    return np.array(perm)


def kernel(x, table, W, b, bn_gamma, bn_beta, bn_mean, bn_var, ln_gamma,
           ln_beta):
    B, L = x.shape
    V, D = table.shape
    s = bn_gamma * lax.rsqrt(bn_var + 1e-3)
    W2 = W * s[None, :] * (1.0 / L)
    b2 = (b - bn_mean) * s + bn_beta
    W2p = W2[_pair_perm(D), :]
    table_pk = _tc_pack_table(jnp.swapaxes(table, 0, 1))
    x2 = _remap_idx(x.astype(jnp.int32))
    h = _sc_gather_sum(x2, table_pk, B, L, D)
    return _tc_head(h, W2p, b2.reshape(1, D), ln_gamma.reshape(1, D),
                    ln_beta.reshape(1, D))
